# baseline (speedup 1.0000x reference)
import functools
import math

import numpy as np
import jax
import jax.numpy as jnp
from jax.experimental import pallas as pl
from jax.experimental.pallas import tpu as pltpu
from jax.experimental.pallas import tpu_sc as plsc

_K = 2
_EPS = 1e-08
_TN = 1024
_INT_MIN = jnp.int32(-2147483648)


def _dot(a, b):
    return jax.lax.dot(a, b, preferred_element_type=jnp.float32)


def _make_sc_topk(n, m, b):
    info = plsc.get_sparse_core_info()
    nw = info.num_cores * info.num_subcores
    tpw = n // nw
    mb = m * b
    mesh = plsc.VectorSubcoreMesh(core_axis_name="c", subcore_axis_name="s")

    @functools.partial(
        pl.kernel,
        mesh=mesh,
        compiler_params=pltpu.CompilerParams(
            use_tc_tiling_on_sc=False, needs_layout_passes=False),
        out_type=[
            jax.ShapeDtypeStruct((n, _K * b), jnp.float32),
            jax.ShapeDtypeStruct((n, _K), jnp.int32),
        ],
        scratch_types=[
            pltpu.VMEM((tpw, mb), jnp.float32),
            pltpu.VMEM((tpw, _K * b), jnp.float32),
            pltpu.VMEM((tpw, _K), jnp.int32),
        ],
    )
    def sc_topk(h_hbm, hs_out, idx_out, h_v, hs_v, idx_v):
        wid = jax.lax.axis_index("s") * info.num_cores + jax.lax.axis_index("c")
        base = wid * tpw
        pltpu.sync_copy(h_hbm.at[pl.ds(base, tpw)], h_v)
        lanes = jax.lax.iota(jnp.int32, 16)
        col0 = lanes * b

        def body(t, carry):
            tv = jnp.full((16,), t, jnp.int32)
            acc = jnp.zeros((16,), jnp.float32)
            for bb in range(b):
                v = plsc.load_gather(h_v, [tv, col0 + bb])
                acc = acc + v * v
            eb = jax.lax.bitcast_convert_type(acc, jnp.int32)
            key = (eb & ~15) | (15 - lanes)
            k0 = jax.lax.reduce_max(key, axes=(0,))
            key2 = jnp.where(key == k0, _INT_MIN, key)
            k1 = jax.lax.reduce_max(key2, axes=(0,))
            m0 = 15 - (k0 & 15)
            m1 = 15 - (k1 & 15)
            hs_v[t, pl.ds(0, 16)] = plsc.load_gather(h_v, [tv, m0 * b + lanes])
            hs_v[t, pl.ds(16, 16)] = plsc.load_gather(
                h_v, [tv, m0 * b + lanes + 16])
            hs_v[t, pl.ds(32, 16)] = plsc.load_gather(
                h_v, [tv, m1 * b + lanes])
            hs_v[t, pl.ds(48, 16)] = plsc.load_gather(
                h_v, [tv, m1 * b + lanes + 16])
            iv = jnp.where(lanes == 0, m0, m1)
            plsc.store_scatter(idx_v, [tv, lanes], iv, mask=lanes < _K)
            return carry

        jax.lax.fori_loop(0, tpw, body, 0)
        pltpu.sync_copy(hs_v, hs_out.at[pl.ds(base, tpw)])
        pltpu.sync_copy(idx_v, idx_out.at[pl.ds(base, tpw)])

    return sc_topk


def _tc_body(n_grid, n_tokens, m_experts, b_code, x_ref, h_ref, vt_ref,
             s_ref, scal_ref):
    g = pl.program_id(0)
    h = h_ref[...]
    hh = h * h
    s_b = s_ref[...]
    hi = hh.astype(jnp.bfloat16)
    r1 = hh - hi.astype(jnp.float32)
    mid = r1.astype(jnp.bfloat16)
    lo = (r1 - mid.astype(jnp.float32)).astype(jnp.bfloat16)
    energy = (_dot(hi, s_b) + _dot(mid, s_b)) + _dot(lo, s_b)

    iota_m = jax.lax.broadcasted_iota(jnp.int32, (h.shape[0], m_experts), 1)
    riota = (m_experts - 1) - iota_m
    e0 = jnp.max(energy, axis=1, keepdims=True)
    m0 = (m_experts - 1) - jnp.max(
        jnp.where(energy == e0, riota, -1), axis=1, keepdims=True)
    masked = jnp.where(iota_m == m0, -jnp.inf, energy)
    e1 = jnp.max(masked, axis=1, keepdims=True)
    m1 = (m_experts - 1) - jnp.max(
        jnp.where(masked == e1, riota, -1), axis=1, keepdims=True)

    mb = m_experts * b_code
    jexp = jax.lax.broadcasted_iota(jnp.int32, (h.shape[0], mb), 1) // b_code
    h_masked = jnp.where((jexp == m0) | (jexp == m1), h, 0.0).astype(
        jnp.bfloat16)
    x_hat = jax.lax.dot_general(
        h_masked, vt_ref[...], (((1,), (1,)), ((), ())),
        preferred_element_type=jnp.float32)
    x = x_ref[...]
    resid = x - x_hat

    cap_s = jnp.sum(jnp.where((iota_m == m0) | (iota_m == m1), energy, 0.0))
    rec_s = jnp.sum(x_hat * x_hat)
    unc_s = jnp.sum(resid * resid)
    esum = jnp.sum(energy, axis=0, keepdims=True)
    il = jax.lax.broadcasted_iota(jnp.int32, (1, 128), 1)
    stepvec = ((il == 0).astype(jnp.float32) * cap_s
               + (il == 1).astype(jnp.float32) * rec_s
               + (il == 2).astype(jnp.float32) * unc_s
               + jnp.concatenate(
                   [jnp.zeros((1, 8), jnp.float32), esum,
                    jnp.zeros((1, 128 - 8 - m_experts), jnp.float32)],
                   axis=1))

    @pl.when(g == 0)
    def _():
        scal_ref[...] = stepvec

    @pl.when(g > 0)
    def _():
        scal_ref[...] = scal_ref[...] + stepvec

    @pl.when(g == n_grid - 1)
    def _():
        acc = scal_ref[...]
        n_f = float(n_tokens)
        emask = ((il >= 8) & (il < 8 + m_experts)).astype(jnp.float32)
        avg = acc * emask / n_f
        denom = jnp.maximum(jnp.sum(avg), _EPS)
        probs = jnp.maximum(avg / denom, _EPS)
        ent = -jnp.sum(emask * probs * jnp.log(probs)) / math.log(m_experts)
        cap = jnp.sum(acc * (il == 0).astype(jnp.float32)) / n_f
        rec = jnp.sum(acc * (il == 1).astype(jnp.float32)) / n_f
        unc = jnp.sum(acc * (il == 2).astype(jnp.float32)) / n_f
        aux = unc + 0.5 * (1.0 - ent)
        scal_ref[...] = ((il == 0).astype(jnp.float32) * cap
                         + (il == 1).astype(jnp.float32) * rec
                         + (il == 2).astype(jnp.float32) * unc
                         + (il == 3).astype(jnp.float32) * ent
                         + (il == 4).astype(jnp.float32) * aux)


@jax.jit
def kernel(x_flat, h_all, V):
    n, d = x_flat.shape
    _, m, b = h_all.shape
    mb = m * b
    h2 = h_all.reshape(n, mb)
    vt = V.reshape(d, mb).astype(jnp.bfloat16)
    tn = min(_TN, n)
    n_grid = n // tn

    j = np.arange(mb)
    s_np = jnp.asarray(
        (j[:, None] // b == np.arange(m)[None, :]).astype(np.float32),
        dtype=jnp.bfloat16)

    hs2d, idx = _make_sc_topk(n, m, b)(h2)

    body = functools.partial(_tc_body, n_grid, n, m, b)
    (scal,) = pl.pallas_call(
        body,
        grid=(n_grid,),
        in_specs=[
            pl.BlockSpec((tn, d), lambda g: (g, 0)),
            pl.BlockSpec((tn, mb), lambda g: (g, 0)),
            pl.BlockSpec((d, mb), lambda g: (0, 0)),
            pl.BlockSpec((mb, m), lambda g: (0, 0)),
        ],
        out_specs=[
            pl.BlockSpec((1, 128), lambda g: (0, 0)),
        ],
        out_shape=[
            jax.ShapeDtypeStruct((1, 128), jnp.float32),
        ],
    )(x_flat, h2, vt, s_np)

    return (hs2d.reshape(n, _K, b), idx, scal[0, 0], scal[0, 1], scal[0, 2],
            scal[0, 3], scal[0, 4])


# speedup vs baseline: 1.0119x; 1.0119x over previous
import functools
import math

import numpy as np
import jax
import jax.numpy as jnp
from jax.experimental import pallas as pl
from jax.experimental.pallas import tpu as pltpu
from jax.experimental.pallas import tpu_sc as plsc

_K = 2
_EPS = 1e-08
_TN = 1024
_INT_MIN = jnp.int32(-2147483648)


def _dot(a, b):
    return jax.lax.dot(a, b, preferred_element_type=jnp.float32)


def _make_sc_topk(n, m, b):
    info = plsc.get_sparse_core_info()
    nw = info.num_cores * info.num_subcores
    tpw = n // nw
    mb = m * b
    mesh = plsc.VectorSubcoreMesh(core_axis_name="c", subcore_axis_name="s")

    @functools.partial(
        pl.kernel,
        mesh=mesh,
        compiler_params=pltpu.CompilerParams(
            use_tc_tiling_on_sc=False, needs_layout_passes=False),
        out_type=[
            jax.ShapeDtypeStruct((n, _K * b), jnp.float32),
            jax.ShapeDtypeStruct((n, _K), jnp.int32),
        ],
        scratch_types=[
            pltpu.VMEM((tpw, mb), jnp.float32),
            pltpu.VMEM((tpw, _K * b), jnp.float32),
            pltpu.VMEM((tpw, _K), jnp.int32),
        ],
    )
    def sc_topk(h_hbm, hs_out, idx_out, h_v, hs_v, idx_v):
        wid = jax.lax.axis_index("s") * info.num_cores + jax.lax.axis_index("c")
        base = wid * tpw
        pltpu.sync_copy(h_hbm.at[pl.ds(base, tpw)], h_v)
        lanes = jax.lax.iota(jnp.int32, 16)
        col0 = lanes * b

        def body(t, carry):
            tv = jnp.full((16,), t, jnp.int32)
            accs = [jnp.zeros((16,), jnp.float32) for _ in range(4)]
            for bb in range(b):
                v = plsc.load_gather(h_v, [tv, col0 + bb])
                accs[bb % 4] = accs[bb % 4] + v * v
            acc = (accs[0] + accs[1]) + (accs[2] + accs[3])
            eb = jax.lax.bitcast_convert_type(acc, jnp.int32)
            key = (eb & ~15) | (15 - lanes)
            k0 = jax.lax.reduce_max(key, axes=(0,))
            key2 = jnp.where(key == k0, _INT_MIN, key)
            k1 = jax.lax.reduce_max(key2, axes=(0,))
            m0 = 15 - (k0 & 15)
            m1 = 15 - (k1 & 15)
            hs_v[t, pl.ds(0, 16)] = plsc.load_gather(h_v, [tv, m0 * b + lanes])
            hs_v[t, pl.ds(16, 16)] = plsc.load_gather(
                h_v, [tv, m0 * b + lanes + 16])
            hs_v[t, pl.ds(32, 16)] = plsc.load_gather(
                h_v, [tv, m1 * b + lanes])
            hs_v[t, pl.ds(48, 16)] = plsc.load_gather(
                h_v, [tv, m1 * b + lanes + 16])
            iv = jnp.where(lanes == 0, m0, m1)
            plsc.store_scatter(idx_v, [tv, lanes], iv, mask=lanes < _K)
            return carry

        jax.lax.fori_loop(0, tpw, body, 0)
        pltpu.sync_copy(hs_v, hs_out.at[pl.ds(base, tpw)])
        pltpu.sync_copy(idx_v, idx_out.at[pl.ds(base, tpw)])

    return sc_topk


def _tc_body(n_grid, n_tokens, m_experts, b_code, x_ref, h_ref, vt_ref,
             s_ref, scal_ref):
    g = pl.program_id(0)
    h = h_ref[...]
    hh = h * h
    s_b = s_ref[...]
    hi = hh.astype(jnp.bfloat16)
    r1 = hh - hi.astype(jnp.float32)
    mid = r1.astype(jnp.bfloat16)
    lo = (r1 - mid.astype(jnp.float32)).astype(jnp.bfloat16)
    energy = (_dot(hi, s_b) + _dot(mid, s_b)) + _dot(lo, s_b)

    iota_m = jax.lax.broadcasted_iota(jnp.int32, (h.shape[0], m_experts), 1)
    riota = (m_experts - 1) - iota_m
    e0 = jnp.max(energy, axis=1, keepdims=True)
    m0 = (m_experts - 1) - jnp.max(
        jnp.where(energy == e0, riota, -1), axis=1, keepdims=True)
    masked = jnp.where(iota_m == m0, -jnp.inf, energy)
    e1 = jnp.max(masked, axis=1, keepdims=True)
    m1 = (m_experts - 1) - jnp.max(
        jnp.where(masked == e1, riota, -1), axis=1, keepdims=True)

    mb = m_experts * b_code
    jexp = jax.lax.broadcasted_iota(jnp.int32, (h.shape[0], mb), 1) // b_code
    h_masked = jnp.where((jexp == m0) | (jexp == m1), h, 0.0).astype(
        jnp.bfloat16)
    x_hat = jax.lax.dot_general(
        h_masked, vt_ref[...], (((1,), (1,)), ((), ())),
        preferred_element_type=jnp.float32)
    x = x_ref[...]
    resid = x - x_hat

    cap_s = jnp.sum(jnp.where((iota_m == m0) | (iota_m == m1), energy, 0.0))
    rec_s = jnp.sum(x_hat * x_hat)
    unc_s = jnp.sum(resid * resid)
    esum = jnp.sum(energy, axis=0, keepdims=True)
    il = jax.lax.broadcasted_iota(jnp.int32, (1, 128), 1)
    stepvec = ((il == 0).astype(jnp.float32) * cap_s
               + (il == 1).astype(jnp.float32) * rec_s
               + (il == 2).astype(jnp.float32) * unc_s
               + jnp.concatenate(
                   [jnp.zeros((1, 8), jnp.float32), esum,
                    jnp.zeros((1, 128 - 8 - m_experts), jnp.float32)],
                   axis=1))

    @pl.when(g == 0)
    def _():
        scal_ref[...] = stepvec

    @pl.when(g > 0)
    def _():
        scal_ref[...] = scal_ref[...] + stepvec

    @pl.when(g == n_grid - 1)
    def _():
        acc = scal_ref[...]
        n_f = float(n_tokens)
        emask = ((il >= 8) & (il < 8 + m_experts)).astype(jnp.float32)
        avg = acc * emask / n_f
        denom = jnp.maximum(jnp.sum(avg), _EPS)
        probs = jnp.maximum(avg / denom, _EPS)
        ent = -jnp.sum(emask * probs * jnp.log(probs)) / math.log(m_experts)
        cap = jnp.sum(acc * (il == 0).astype(jnp.float32)) / n_f
        rec = jnp.sum(acc * (il == 1).astype(jnp.float32)) / n_f
        unc = jnp.sum(acc * (il == 2).astype(jnp.float32)) / n_f
        aux = unc + 0.5 * (1.0 - ent)
        scal_ref[...] = ((il == 0).astype(jnp.float32) * cap
                         + (il == 1).astype(jnp.float32) * rec
                         + (il == 2).astype(jnp.float32) * unc
                         + (il == 3).astype(jnp.float32) * ent
                         + (il == 4).astype(jnp.float32) * aux)


@jax.jit
def kernel(x_flat, h_all, V):
    n, d = x_flat.shape
    _, m, b = h_all.shape
    mb = m * b
    h2 = h_all.reshape(n, mb)
    vt = V.reshape(d, mb).astype(jnp.bfloat16)
    tn = min(_TN, n)
    n_grid = n // tn

    j = np.arange(mb)
    s_np = jnp.asarray(
        (j[:, None] // b == np.arange(m)[None, :]).astype(np.float32),
        dtype=jnp.bfloat16)

    hs2d, idx = _make_sc_topk(n, m, b)(h2)

    body = functools.partial(_tc_body, n_grid, n, m, b)
    (scal,) = pl.pallas_call(
        body,
        grid=(n_grid,),
        in_specs=[
            pl.BlockSpec((tn, d), lambda g: (g, 0)),
            pl.BlockSpec((tn, mb), lambda g: (g, 0)),
            pl.BlockSpec((d, mb), lambda g: (0, 0)),
            pl.BlockSpec((mb, m), lambda g: (0, 0)),
        ],
        out_specs=[
            pl.BlockSpec((1, 128), lambda g: (0, 0)),
        ],
        out_shape=[
            jax.ShapeDtypeStruct((1, 128), jnp.float32),
        ],
    )(x_flat, h2, vt, s_np)

    return (hs2d.reshape(n, _K, b), idx, scal[0, 0], scal[0, 1], scal[0, 2],
            scal[0, 3], scal[0, 4])
